# Initial kernel scaffold; baseline (speedup 1.0000x reference)
import jax
import jax.numpy as jnp
from jax.experimental import pallas as pl


def kernel(x, adj, W1, W2, Wfc, bfc, gamma, beta, running_mean, running_var):
    raise NotImplementedError("write your pallas kernel here")


# speedup vs baseline: 1.3947x; 1.3947x over previous
import jax
import jax.numpy as jnp
from jax.experimental import pallas as pl

_N, _D, _H = 10000, 128, 16
_BM = 400


def _xw_kernel(x_ref, w_ref, out_ref):
    out_ref[...] = jnp.dot(x_ref[...], w_ref[...],
                           preferred_element_type=jnp.float32)


def _gc_kernel(adj_ref, xw_ref, wfc_ref, aff_ref, mu_ref, lv_ref, xrec_ref):
    t = jnp.dot(adj_ref[...], xw_ref[...],
                preferred_element_type=jnp.float32)
    t = jnp.where(t >= 0, t, 0.01 * t)
    mu = t[:, :_H]
    mu_ref[...] = mu
    lv_ref[...] = t[:, _H:]
    h = jnp.dot(mu, wfc_ref[...], preferred_element_type=jnp.float32)
    xrec_ref[...] = h * aff_ref[0:1, :] + aff_ref[1:2, :]


def _ip_kernel(zb_ref, z_ref, out_ref):
    out_ref[...] = jax.lax.dot_general(
        zb_ref[...], z_ref[...], (((1,), (1,)), ((), ())),
        preferred_element_type=jnp.float32)


def kernel(x, adj, W1, W2, Wfc, bfc, gamma, beta, running_mean, running_var):
    n, d = x.shape
    h = W1.shape[1]
    nb = n // _BM

    wcat = jnp.concatenate([W1, W2], axis=1)

    xw = pl.pallas_call(
        _xw_kernel,
        grid=(1,),
        in_specs=[
            pl.BlockSpec((n, d), lambda i: (0, 0)),
            pl.BlockSpec((d, 2 * h), lambda i: (0, 0)),
        ],
        out_specs=pl.BlockSpec((n, 2 * h), lambda i: (0, 0)),
        out_shape=jax.ShapeDtypeStruct((n, 2 * h), jnp.float32),
    )(x, wcat)

    scale = gamma * jax.lax.rsqrt(running_var + 1e-5)
    shift = (bfc - running_mean) * scale + beta
    aff = jnp.stack([scale, shift], axis=0)

    mu, logvar, x_rec = pl.pallas_call(
        _gc_kernel,
        grid=(nb,),
        in_specs=[
            pl.BlockSpec((_BM, n), lambda i: (i, 0)),
            pl.BlockSpec((n, 2 * h), lambda i: (0, 0)),
            pl.BlockSpec((h, d), lambda i: (0, 0)),
            pl.BlockSpec((2, d), lambda i: (0, 0)),
        ],
        out_specs=[
            pl.BlockSpec((_BM, h), lambda i: (i, 0)),
            pl.BlockSpec((_BM, h), lambda i: (i, 0)),
            pl.BlockSpec((_BM, d), lambda i: (i, 0)),
        ],
        out_shape=[
            jax.ShapeDtypeStruct((n, h), jnp.float32),
            jax.ShapeDtypeStruct((n, h), jnp.float32),
            jax.ShapeDtypeStruct((n, d), jnp.float32),
        ],
    )(adj, xw, Wfc, aff)

    adj_rec = pl.pallas_call(
        _ip_kernel,
        grid=(nb,),
        in_specs=[
            pl.BlockSpec((_BM, h), lambda i: (i, 0)),
            pl.BlockSpec((n, h), lambda i: (0, 0)),
        ],
        out_specs=pl.BlockSpec((_BM, n), lambda i: (i, 0)),
        out_shape=jax.ShapeDtypeStruct((n, n), jnp.float32),
    )(mu, mu)

    z = mu
    return (adj_rec, mu, logvar, z, x_rec)


# speedup vs baseline: 1.4229x; 1.0202x over previous
import jax
import jax.numpy as jnp
from jax.experimental import pallas as pl
from jax.experimental.pallas import tpu as pltpu

_N, _D, _H = 10000, 128, 16
_BM = 200
_NB = _N // _BM


def _mega_kernel(adj_ref, x_ref, wcat_ref, wfc_ref, aff_ref,
                 mu_ref, lv_ref, xrec_ref, rec_ref,
                 xw_s, z_s):
    s = pl.program_id(0)

    @pl.when(s == 0)
    def _xw_phase():
        xw_s[...] = jnp.dot(x_ref[...], wcat_ref[...],
                            preferred_element_type=jnp.float32)

    @pl.when((s >= 1) & (s <= _NB))
    def _gc_phase():
        t = jnp.dot(adj_ref[...], xw_s[...],
                    preferred_element_type=jnp.float32)
        t = jnp.where(t >= 0, t, 0.01 * t)
        mu = t[:, :_H]
        mu_ref[...] = mu
        lv_ref[...] = t[:, _H:]
        z_s[pl.ds((s - 1) * _BM, _BM), :] = mu
        h = jnp.dot(mu, wfc_ref[...], preferred_element_type=jnp.float32)
        xrec_ref[...] = h * aff_ref[0:1, :] + aff_ref[1:2, :]

    @pl.when(s > _NB)
    def _ip_phase():
        zb = z_s[pl.ds((s - _NB - 1) * _BM, _BM), :]
        rec_ref[...] = jax.lax.dot_general(
            zb, z_s[...], (((1,), (1,)), ((), ())),
            preferred_element_type=jnp.float32)


def kernel(x, adj, W1, W2, Wfc, bfc, gamma, beta, running_mean, running_var):
    n, d = x.shape
    h = W1.shape[1]

    wcat = jnp.concatenate([W1, W2], axis=1)
    scale = gamma * jax.lax.rsqrt(running_var + 1e-5)
    shift = (bfc - running_mean) * scale + beta
    aff = jnp.stack([scale, shift], axis=0)

    gc_idx = lambda s: (jnp.clip(s - 1, 0, _NB - 1), 0)
    ip_idx = lambda s: (jnp.clip(s - _NB - 1, 0, _NB - 1), 0)

    mu, logvar, x_rec, adj_rec = pl.pallas_call(
        _mega_kernel,
        grid=(1 + 2 * _NB,),
        in_specs=[
            pl.BlockSpec((_BM, n), gc_idx),
            pl.BlockSpec((n, d), lambda s: (0, 0)),
            pl.BlockSpec((d, 2 * h), lambda s: (0, 0)),
            pl.BlockSpec((h, d), lambda s: (0, 0)),
            pl.BlockSpec((2, d), lambda s: (0, 0)),
        ],
        out_specs=[
            pl.BlockSpec((_BM, h), gc_idx),
            pl.BlockSpec((_BM, h), gc_idx),
            pl.BlockSpec((_BM, d), gc_idx),
            pl.BlockSpec((_BM, n), ip_idx),
        ],
        out_shape=[
            jax.ShapeDtypeStruct((n, h), jnp.float32),
            jax.ShapeDtypeStruct((n, h), jnp.float32),
            jax.ShapeDtypeStruct((n, d), jnp.float32),
            jax.ShapeDtypeStruct((n, n), jnp.float32),
        ],
        scratch_shapes=[
            pltpu.VMEM((n, 2 * h), jnp.float32),
            pltpu.VMEM((n, h), jnp.float32),
        ],
    )(adj, x, wcat, Wfc, aff)

    z = mu
    return (adj_rec, mu, logvar, z, x_rec)
